# baseline (speedup 1.0000x reference)
import jax
import jax.numpy as jnp
from jax.experimental import pallas as pl
from jax.experimental.pallas import tpu as pltpu


def _copy_kernel(x_ref, o_ref):
    o_ref[...] = x_ref[...]


def kernel(x, edge_index, edge_attr, batch):
    n, d = x.shape
    return pl.pallas_call(
        _copy_kernel,
        grid=(4,),
        in_specs=[pl.BlockSpec((n // 4, d), lambda i: (i, 0))],
        out_specs=pl.BlockSpec((n // 4, d), lambda i: (i, 0)),
        compiler_params=pltpu.CompilerParams(dimension_semantics=("parallel",)),
        out_shape=jax.ShapeDtypeStruct(x.shape, x.dtype),
    )(x)


# speedup vs baseline: 1.4158x; 1.4158x over previous
import jax
import jax.numpy as jnp
from jax.experimental import pallas as pl


def _copy_kernel(x_ref, o_ref):
    o_ref[...] = x_ref[...]


def kernel(x, edge_index, edge_attr, batch):
    n, d = x.shape
    return pl.pallas_call(
        _copy_kernel,
        grid=(2,),
        in_specs=[pl.BlockSpec((n // 2, d), lambda i: (i, 0))],
        out_specs=pl.BlockSpec((n // 2, d), lambda i: (i, 0)),
        out_shape=jax.ShapeDtypeStruct(x.shape, x.dtype),
    )(x)
